# Initial kernel scaffold; baseline (speedup 1.0000x reference)
import jax
import jax.numpy as jnp
from jax.experimental import pallas as pl


def kernel(in_feature, adj, W0, b0, W1, b1):
    raise NotImplementedError("write your pallas kernel here")


# speedup vs baseline: 1.0102x; 1.0102x over previous
import functools

import jax
import jax.numpy as jnp
from jax.experimental import pallas as pl

_N = 10000
_D = 128
_BM_BIG = 400
_BM_SMALL = 2000
_G = _N // _BM_BIG


def _small_matmul_kernel(x_ref, w_ref, out_ref):
    out_ref[...] = jnp.dot(x_ref[...], w_ref[...],
                           preferred_element_type=jnp.float32)


def _big_layer_kernel(adj_ref, s_ref, b_ref, a_ref, cs_ref):
    t = jnp.dot(adj_ref[...], s_ref[...], preferred_element_type=jnp.float32)
    a = jnp.maximum(t + b_ref[...], 0.0)
    a_ref[...] = a
    cs_ref[...] = jnp.sum(a, axis=0).reshape(1, 1, _D)


def _pn_matmul_kernel(a_ref, cs_ref, w_ref, out_ref):
    mean = jnp.sum(cs_ref[...], axis=(0, 1)) * (1.0 / _N)
    x = a_ref[...] - mean[None, :]
    rn = jax.lax.rsqrt(1e-6 + jnp.sum(x * x, axis=1, keepdims=True))
    out_ref[...] = jnp.dot(x * rn, w_ref[...],
                           preferred_element_type=jnp.float32)


def _pn_kernel(a_ref, cs_ref, out_ref):
    mean = jnp.sum(cs_ref[...], axis=(0, 1)) * (1.0 / _N)
    x = a_ref[...] - mean[None, :]
    rn = jax.lax.rsqrt(1e-6 + jnp.sum(x * x, axis=1, keepdims=True))
    out_ref[...] = x * rn


def _small_matmul(x, w):
    return pl.pallas_call(
        _small_matmul_kernel,
        grid=(_N // _BM_SMALL,),
        in_specs=[
            pl.BlockSpec((_BM_SMALL, _D), lambda i: (i, 0)),
            pl.BlockSpec((_D, _D), lambda i: (0, 0)),
        ],
        out_specs=pl.BlockSpec((_BM_SMALL, _D), lambda i: (i, 0)),
        out_shape=jax.ShapeDtypeStruct((_N, _D), jnp.float32),
    )(x, w)


def _big_layer(adj, s, b):
    return pl.pallas_call(
        _big_layer_kernel,
        grid=(_G,),
        in_specs=[
            pl.BlockSpec((_BM_BIG, _N), lambda i: (i, 0)),
            pl.BlockSpec((_N, _D), lambda i: (0, 0)),
            pl.BlockSpec((1, _D), lambda i: (0, 0)),
        ],
        out_specs=[
            pl.BlockSpec((_BM_BIG, _D), lambda i: (i, 0)),
            pl.BlockSpec((1, 1, _D), lambda i: (i, 0, 0)),
        ],
        out_shape=[
            jax.ShapeDtypeStruct((_N, _D), jnp.float32),
            jax.ShapeDtypeStruct((_G, 1, _D), jnp.float32),
        ],
    )(adj, s, b)


def _pn_matmul(a, cs, w):
    return pl.pallas_call(
        _pn_matmul_kernel,
        grid=(_N // _BM_SMALL,),
        in_specs=[
            pl.BlockSpec((_BM_SMALL, _D), lambda i: (i, 0)),
            pl.BlockSpec((_G, 1, _D), lambda i: (0, 0, 0)),
            pl.BlockSpec((_D, _D), lambda i: (0, 0)),
        ],
        out_specs=pl.BlockSpec((_BM_SMALL, _D), lambda i: (i, 0)),
        out_shape=jax.ShapeDtypeStruct((_N, _D), jnp.float32),
    )(a, cs, w)


def _pn(a, cs):
    return pl.pallas_call(
        _pn_kernel,
        grid=(_N // _BM_SMALL,),
        in_specs=[
            pl.BlockSpec((_BM_SMALL, _D), lambda i: (i, 0)),
            pl.BlockSpec((_G, 1, _D), lambda i: (0, 0, 0)),
        ],
        out_specs=pl.BlockSpec((_BM_SMALL, _D), lambda i: (i, 0)),
        out_shape=jax.ShapeDtypeStruct((_N, _D), jnp.float32),
    )(a, cs)


@jax.jit
def kernel(in_feature, adj, W0, b0, W1, b1):
    s0 = _small_matmul(in_feature, W0)
    a0, cs0 = _big_layer(adj, s0, b0.reshape(1, _D))
    s1 = _pn_matmul(a0, cs0, W1)
    a1, cs1 = _big_layer(adj, s1, b1.reshape(1, _D))
    return _pn(a1, cs1)
